# baseline (speedup 1.0000x reference)
import jax
import jax.numpy as jnp
from jax import lax
from jax.experimental import pallas as pl
from jax.experimental.pallas import tpu as pltpu
from jax.experimental.pallas import tpu_sc as plsc

NC = 2
NS = 16
LANE = 16
CH = 128


def _sc_mesh():
    return plsc.VectorSubcoreMesh(
        core_axis_name="c", subcore_axis_name="s", num_cores=NC, num_subcores=NS)


def _make_embed_gather(Nvpad, D):
    NW = NC * NS
    KV = Nvpad // (NW * CH)
    rows_max = Nvpad // CH - 1

    def body(id_emb, label_emb, vidm, labm, out, vidx, lidx, gbuf, lbuf, obuf,
             si0, si1, si2, sg0, sg1, sg2, sw0, sw1, sw2):
        semi = (si0, si1, si2)
        semg = (sg0, sg1, sg2)
        semw = (sw0, sw1, sw2)
        c = lax.axis_index("c")
        s = lax.axis_index("s")
        wid = c * NS + s
        t0 = wid * KV

        def idx_load(q, r):
            pltpu.async_copy(vidm.at[r], vidx.at[q], semi[q])
            pltpu.async_copy(labm.at[r], lidx.at[q], semi[q])

        def idx_wait(q):
            pltpu.make_async_copy(vidm.at[t0], vidx.at[q], semi[q]).wait()
            pltpu.make_async_copy(labm.at[t0], lidx.at[q], semi[q]).wait()

        def gathers(q):
            pltpu.async_copy(id_emb.at[vidx.at[q]], gbuf.at[q], semg[q])
            pltpu.async_copy(label_emb.at[lidx.at[q]], lbuf.at[q], semg[q])

        def gathers_wait(q):
            pltpu.make_async_copy(id_emb.at[vidx.at[q]], gbuf.at[q],
                                  semg[q]).wait()
            pltpu.make_async_copy(label_emb.at[lidx.at[q]], lbuf.at[q],
                                  semg[q]).wait()

        def wout_wait(q):
            pltpu.make_async_copy(obuf.at[q], out.at[pl.ds(Nvpad, CH)],
                                  semw[q]).wait()

        pltpu.sync_copy(vidm.at[t0], vidx.at[0])
        pltpu.sync_copy(labm.at[t0], lidx.at[0])
        gathers(0)
        idx_load(1, t0 + 1)
        for q in range(3):
            pltpu.async_copy(obuf.at[q], out.at[pl.ds(Nvpad, CH)], semw[q])

        def block(ib, carry):
            ibase = ib * 3
            for k in range(3):
                i = ibase + k
                q, q1, q2 = k, (k + 1) % 3, (k + 2) % 3
                gathers_wait(q)
                idx_wait(q1)
                gathers(q1)
                wout_wait(q)
                for jj in range(CH):
                    obuf[q, jj, :] = gbuf[q, jj, :] + lbuf[q, jj, :]
                pltpu.async_copy(obuf.at[q], out.at[pl.ds((t0 + i) * CH, CH)],
                                 semw[q])
                idx_load(q2, jnp.minimum(t0 + i + 2, rows_max))
            return carry

        lax.fori_loop(0, KV // 3, block, 0)

        gathers_wait(KV % 3)
        idx_wait((KV + 1) % 3)
        for q in range(3):
            wout_wait(q)

    return pl.kernel(
        body,
        out_type=jax.ShapeDtypeStruct((Nvpad + CH, D), jnp.float32),
        mesh=_sc_mesh(),
        compiler_params=pltpu.CompilerParams(use_tc_tiling_on_sc=False),
        scratch_types=[
            pltpu.VMEM((3, CH), jnp.int32),
            pltpu.VMEM((3, CH), jnp.int32),
            pltpu.VMEM((3, CH, D), jnp.float32),
            pltpu.VMEM((3, CH, D), jnp.float32),
            pltpu.VMEM((3, CH, D), jnp.float32),
            pltpu.SemaphoreType.DMA,
            pltpu.SemaphoreType.DMA,
            pltpu.SemaphoreType.DMA,
            pltpu.SemaphoreType.DMA,
            pltpu.SemaphoreType.DMA,
            pltpu.SemaphoreType.DMA,
            pltpu.SemaphoreType.DMA,
            pltpu.SemaphoreType.DMA,
            pltpu.SemaphoreType.DMA,
        ],
    )


EDGE_G = 4


def _make_edge_agg(Epad, Npad, D):
    NW = NC * NS
    G = EDGE_G
    KPT = Epad // (NW * CH)
    NG = KPT // G
    NB = NG // 3
    rows_total = Epad // CH
    NPT = Npad // NS
    ZR = CH
    nz_full, nz_tail = divmod(NPT, ZR)

    def body(h, srcm, dstm, out, agg, sidx, didx, gbuf, zbuf,
             si0, si1, si2, sg0, sg1, sg2, ss0, ss1, ss2):
        semi = (si0, si1, si2)
        semg = (sg0, sg1, sg2)
        sems = (ss0, ss1, ss2)
        c = lax.axis_index("c")
        s = lax.axis_index("s")
        wid = c * NS + s
        zero = jnp.zeros((LANE,), jnp.float32)
        for i in range(ZR):
            zbuf[i, :] = zero
        row0 = s * NPT
        for k in range(nz_full):
            pltpu.async_copy(zbuf, agg.at[pl.ds(row0 + k * ZR, ZR)], si0)
        if nz_tail:
            pltpu.async_copy(zbuf.at[pl.ds(0, nz_tail)],
                            agg.at[pl.ds(row0 + nz_full * ZR, nz_tail)], si0)
        for k in range(nz_full):
            pltpu.make_async_copy(zbuf, agg.at[pl.ds(row0, ZR)], si0).wait()
        if nz_tail:
            pltpu.make_async_copy(zbuf.at[pl.ds(0, nz_tail)],
                                  agg.at[pl.ds(row0, nz_tail)], si0).wait()
        plsc.subcore_barrier()

        t0 = wid * KPT
        last = rows_total - G

        def gather(q, j2):
            return pltpu.async_copy(h.at[sidx.at[q].at[j2]],
                                    gbuf.at[q].at[j2], semg[q])

        def gather_wait(q, j2):
            pltpu.make_async_copy(h.at[sidx.at[q].at[j2]],
                                  gbuf.at[q].at[j2], semg[q]).wait()

        def scatter(q, j2):
            return pltpu.async_copy(gbuf.at[q].at[j2],
                                    agg.at[didx.at[q].at[j2]], sems[q],
                                    add=True)

        def scatter_wait(q, j2):
            pltpu.make_async_copy(gbuf.at[q].at[j2],
                                  agg.at[didx.at[q].at[j2]], sems[q]).wait()

        def idx_load(q, off):
            pltpu.async_copy(srcm.at[pl.ds(off, G)], sidx.at[q], semi[q])
            pltpu.async_copy(dstm.at[pl.ds(off, G)], didx.at[q], semi[q])

        def idx_wait(q):
            pltpu.make_async_copy(srcm.at[pl.ds(t0, G)], sidx.at[q],
                                  semi[q]).wait()
            pltpu.make_async_copy(dstm.at[pl.ds(t0, G)], didx.at[q],
                                  semi[q]).wait()

        pltpu.sync_copy(srcm.at[pl.ds(t0, G)], sidx.at[0])
        pltpu.sync_copy(dstm.at[pl.ds(t0, G)], didx.at[0])
        for j2 in range(G):
            gather(0, j2)
        idx_load(1, t0 + G)
        for j2 in range(G):
            pltpu.async_copy(zbuf, agg.at[didx.at[0].at[j2]], sems[2],
                             add=True)

        def block(ib, carry):
            jbase = ib * 3
            for k in range(3):
                jj = jbase + k
                q, q1, q2 = k, (k + 1) % 3, (k + 2) % 3
                for j2 in range(G):
                    gather_wait(q, j2)
                for j2 in range(G):
                    scatter(q, j2)
                idx_wait(q1)
                for j2 in range(G):
                    scatter_wait(q2, j2)
                for j2 in range(G):
                    gather(q1, j2)
                idx_load(q2, jnp.minimum(t0 + (jj + 2) * G, last))
            return carry

        lax.fori_loop(0, NB, block, 0)

        qlast = (NG - 1) % 3
        for j2 in range(G):
            scatter_wait(qlast, j2)
        for j2 in range(G):
            gather_wait(NG % 3, j2)
        idx_wait((NG + 1) % 3)

        plsc.subcore_barrier()
        pltpu.sync_copy(agg.at[pl.ds(row0, NPT)], out.at[c, pl.ds(row0, NPT)])

    return pl.kernel(
        body,
        out_type=jax.ShapeDtypeStruct((NC, Npad, D), jnp.float32),
        mesh=_sc_mesh(),
        compiler_params=pltpu.CompilerParams(use_tc_tiling_on_sc=False),
        scratch_types=[
            pltpu.VMEM_SHARED((Npad, D), jnp.float32),
            pltpu.VMEM((3, G, CH), jnp.int32),
            pltpu.VMEM((3, G, CH), jnp.int32),
            pltpu.VMEM((3, G, CH, D), jnp.float32),
            pltpu.VMEM((ZR, D), jnp.float32),
            pltpu.SemaphoreType.DMA,
            pltpu.SemaphoreType.DMA,
            pltpu.SemaphoreType.DMA,
            pltpu.SemaphoreType.DMA,
            pltpu.SemaphoreType.DMA,
            pltpu.SemaphoreType.DMA,
            pltpu.SemaphoreType.DMA,
            pltpu.SemaphoreType.DMA,
            pltpu.SemaphoreType.DMA,
        ],
    )


def _embed_tc(xg_p, deg8, e8, a_avg, degW_t, degb_t, lng_t, lnb_t, NP, BP):
    def body(xg_ref, d_ref, e8_ref, av_ref, w_ref, b_ref, g_ref, bb_ref, o_ref):
        dl = jnp.log1p(jnp.clip(d_ref[...], 0.0, 1e6))
        dexp = jnp.dot(dl, e8_ref[...], preferred_element_type=jnp.float32)
        x = xg_ref[...] + dexp * w_ref[...] + b_ref[...]
        av = av_ref[...]
        m = jnp.dot(x, av, preferred_element_type=jnp.float32)
        xc = x - m
        v = jnp.dot(xc * xc, av, preferred_element_type=jnp.float32)
        y = xc / jnp.sqrt(v + 1e-5) * g_ref[...] + bb_ref[...]
        o_ref[...] = jax.nn.gelu(y)

    row = pl.BlockSpec((BP, 128), lambda i: (i, 0))
    vec = pl.BlockSpec((1, 128), lambda i: (0, 0))
    return pl.pallas_call(
        body,
        grid=(NP // BP,),
        in_specs=[row, pl.BlockSpec((BP, 8), lambda i: (i, 0)),
                  pl.BlockSpec((8, 128), lambda i: (0, 0)),
                  pl.BlockSpec((128, 128), lambda i: (0, 0)), vec, vec, vec, vec],
        out_specs=row,
        out_shape=jax.ShapeDtypeStruct((NP, 128), jnp.float32),
    )(xg_p, deg8, e8, a_avg, degW_t, degb_t, lng_t, lnb_t)


def _gin_tc(h_p, aggpair_p, W1b, b1t, W2b, b2t, eps, NP, BP, N, final=None):
    def mlp(h_ref, agg_ref, w1, b1r, w2, b2r, eps_ref):
        hh = h_ref[...]
        agg = agg_ref[0] + agg_ref[1]
        z = (1.0 + eps_ref[0, 0]) * hh + agg
        z = jax.nn.gelu(jnp.dot(z, w1[...], preferred_element_type=jnp.float32)
                        + b1r[...])
        z = jnp.dot(z, w2[...], preferred_element_type=jnp.float32) + b2r[...]
        return z + hh

    row = pl.BlockSpec((BP, 128), lambda i: (i, 0))
    vec = pl.BlockSpec((1, 128), lambda i: (0, 0))
    mat = pl.BlockSpec((128, 128), lambda i: (0, 0))
    agg_spec = pl.BlockSpec((2, BP, 128), lambda i: (0, i, 0))
    scal = pl.BlockSpec((1, 1), lambda i: (0, 0))

    if final is None:
        def body(h_ref, agg_ref, w1, b1r, w2, b2r, eps_ref, o_ref):
            o_ref[...] = mlp(h_ref, agg_ref, w1, b1r, w2, b2r, eps_ref)

        return pl.pallas_call(
            body,
            grid=(NP // BP,),
            in_specs=[row, agg_spec, mat, vec, mat, vec, scal],
            out_specs=row,
            out_shape=jax.ShapeDtypeStruct((NP, 128), jnp.float32),
        )(h_p, aggpair_p, W1b, b1t, W2b, b2t, eps.reshape(1, 1))

    embed_p, alpha, pool_scale = final

    def body(h_ref, agg_ref, w1, b1r, w2, b2r, eps_ref, ex_ref, al_ref, ps_ref,
             o_ref):
        h2 = mlp(h_ref, agg_ref, w1, b1r, w2, b2r, eps_ref)
        jk = h_ref[...] + h2
        gate = jax.nn.sigmoid(al_ref[0, 0])
        out = gate * jk + (1.0 - gate) * ex_ref[...]
        o_ref[...] = out * jax.nn.softplus(ps_ref[0, 0])

    return pl.pallas_call(
        body,
        grid=(NP // BP,),
        in_specs=[row, agg_spec, mat, vec, mat, vec, scal, row, scal, scal],
        out_specs=row,
        out_shape=jax.ShapeDtypeStruct((NP, 128), jnp.float32),
    )(h_p, aggpair_p, W1b, b1t, W2b, b2t, eps.reshape(1, 1), embed_p,
      alpha.reshape(1, 1), pool_scale.reshape(1, 1))


def kernel(vertex_ids, labels, degree, edge_index, id_emb, label_emb, deg_W,
           deg_b, ln_g, ln_b, W1_0, b1_0, W2_0, b2_0, eps_0, W1_1, b1_1, W2_1,
           b2_1, eps_1, alpha, pool_scale):
    N, D = id_emb.shape
    L = label_emb.shape[0]
    E = edge_index.shape[1]
    NW = NC * NS
    unit_v = NW * CH * 3
    Nvpad = ((N + unit_v - 1) // unit_v) * unit_v
    NP = Nvpad // 8
    BP = NP // 8

    i8 = jnp.eye(8, dtype=jnp.float32)
    W1b_0 = jnp.kron(i8, W1_0)
    W2b_0 = jnp.kron(i8, W2_0)
    W1b_1 = jnp.kron(i8, W1_1)
    W2b_1 = jnp.kron(i8, W2_1)
    b1t_0 = jnp.tile(b1_0, 8).reshape(1, 128)
    b2t_0 = jnp.tile(b2_0, 8).reshape(1, 128)
    b1t_1 = jnp.tile(b1_1, 8).reshape(1, 128)
    b2t_1 = jnp.tile(b2_1, 8).reshape(1, 128)
    lng_t = jnp.tile(ln_g, 8).reshape(1, 128)
    lnb_t = jnp.tile(ln_b, 8).reshape(1, 128)
    degW_t = jnp.tile(deg_W, 8).reshape(1, 128)
    degb_t = jnp.tile(deg_b, 8).reshape(1, 128)
    a_avg = jnp.kron(i8, jnp.full((D, D), 1.0 / D, jnp.float32))
    e8 = jnp.kron(i8, jnp.ones((1, D), jnp.float32))
    deg8 = jnp.concatenate(
        [degree, jnp.zeros((Nvpad - N,), jnp.float32)]).reshape(NP, 8)

    padv = Nvpad - N
    fill = jnp.arange(padv, dtype=jnp.int32)
    vidm = jnp.concatenate([vertex_ids.astype(jnp.int32), fill % N]).reshape(-1, CH)
    labm = jnp.concatenate([labels.astype(jnp.int32), fill % L]).reshape(-1, CH)
    xg = _make_embed_gather(Nvpad, D)(id_emb, label_emb, vidm, labm)
    xg_p = xg[:Nvpad].reshape(-1, 128)

    embed_p = _embed_tc(xg_p, deg8, e8, a_avg, degW_t, degb_t, lng_t, lnb_t,
                        NP, BP)

    unit_e = NW * CH * (3 * EDGE_G)
    Epad = ((E + unit_e - 1) // unit_e) * unit_e
    pade = Epad - E
    trash = 16
    Npad = N + trash
    fe = jnp.arange(pade, dtype=jnp.int32)
    srcm = jnp.concatenate([edge_index[0].astype(jnp.int32), fe % N]).reshape(-1, CH)
    dstm = jnp.concatenate([edge_index[1].astype(jnp.int32), N + fe % trash]).reshape(-1, CH)

    edge_agg = _make_edge_agg(Epad, Npad, D)

    agg0_p = edge_agg(embed_p.reshape(-1, D), srcm, dstm).reshape(NC, -1, 128)
    h1_p = _gin_tc(embed_p, agg0_p, W1b_0, b1t_0, W2b_0, b2t_0, eps_0, NP, BP, N)

    agg1_p = edge_agg(h1_p.reshape(-1, D), srcm, dstm).reshape(NC, -1, 128)
    out_p = _gin_tc(h1_p, agg1_p, W1b_1, b1t_1, W2b_1, b2t_1, eps_1, NP, BP, N,
                    final=(embed_p, alpha, pool_scale))
    return out_p.reshape(-1, D)[:N]


# speedup vs baseline: 1.1240x; 1.1240x over previous
import jax
import jax.numpy as jnp
from jax import lax
from jax.experimental import pallas as pl
from jax.experimental.pallas import tpu as pltpu
from jax.experimental.pallas import tpu_sc as plsc

NC = 2
NS = 16
LANE = 16
CH = 128


def _sc_mesh():
    return plsc.VectorSubcoreMesh(
        core_axis_name="c", subcore_axis_name="s", num_cores=NC, num_subcores=NS)


def _make_embed_gather(Nvpad, D):
    NW = NC * NS
    KV = Nvpad // (NW * CH)
    rows_max = Nvpad // CH - 1

    PR = CH // 8

    def body(id_emb, label_emb, vidm, labm, out, vidx, lidx, gbuf, lbuf, obuf,
             si0, si1, si2, sg0, sg1, sg2, sw0, sw1, sw2):
        semi = (si0, si1, si2)
        semg = (sg0, sg1, sg2)
        semw = (sw0, sw1, sw2)
        c = lax.axis_index("c")
        s = lax.axis_index("s")
        wid = c * NS + s
        t0 = wid * KV

        def idx_load(q, r):
            pltpu.async_copy(vidm.at[r], vidx.at[q], semi[q])
            pltpu.async_copy(labm.at[r], lidx.at[q], semi[q])

        def idx_wait(q):
            pltpu.make_async_copy(vidm.at[t0], vidx.at[q], semi[q]).wait()
            pltpu.make_async_copy(labm.at[t0], lidx.at[q], semi[q]).wait()

        def gathers(q):
            pltpu.async_copy(id_emb.at[vidx.at[q]], gbuf.at[q], semg[q])
            pltpu.async_copy(label_emb.at[lidx.at[q]], lbuf.at[q], semg[q])

        def gathers_wait(q):
            pltpu.make_async_copy(id_emb.at[vidx.at[q]], gbuf.at[q],
                                  semg[q]).wait()
            pltpu.make_async_copy(label_emb.at[lidx.at[q]], lbuf.at[q],
                                  semg[q]).wait()

        trash0 = Nvpad // 8

        def wout_wait(q):
            pltpu.make_async_copy(obuf.at[q], out.at[pl.ds(trash0, PR)],
                                  semw[q]).wait()

        pltpu.sync_copy(vidm.at[t0], vidx.at[0])
        pltpu.sync_copy(labm.at[t0], lidx.at[0])
        gathers(0)
        idx_load(1, t0 + 1)
        for q in range(3):
            pltpu.async_copy(obuf.at[q], out.at[pl.ds(trash0, PR)], semw[q])

        def block(ib, carry):
            ibase = ib * 3
            for k in range(3):
                i = ibase + k
                q, q1, q2 = k, (k + 1) % 3, (k + 2) % 3
                gathers_wait(q)
                idx_wait(q1)
                gathers(q1)
                wout_wait(q)
                for jj in range(CH):
                    obuf[q, jj // 8, pl.ds((jj % 8) * LANE, LANE)] = (
                        gbuf[q, jj, :] + lbuf[q, jj, :])
                pltpu.async_copy(obuf.at[q],
                                 out.at[pl.ds((t0 + i) * PR, PR)], semw[q])
                idx_load(q2, jnp.minimum(t0 + i + 2, rows_max))
            return carry

        lax.fori_loop(0, KV // 3, block, 0)

        gathers_wait(KV % 3)
        idx_wait((KV + 1) % 3)
        for q in range(3):
            wout_wait(q)

    return pl.kernel(
        body,
        out_type=jax.ShapeDtypeStruct((Nvpad // 8 + CH // 8, 128), jnp.float32),
        mesh=_sc_mesh(),
        compiler_params=pltpu.CompilerParams(use_tc_tiling_on_sc=False),
        scratch_types=[
            pltpu.VMEM((3, CH), jnp.int32),
            pltpu.VMEM((3, CH), jnp.int32),
            pltpu.VMEM((3, CH, D), jnp.float32),
            pltpu.VMEM((3, CH, D), jnp.float32),
            pltpu.VMEM((3, CH // 8, 128), jnp.float32),
            pltpu.SemaphoreType.DMA,
            pltpu.SemaphoreType.DMA,
            pltpu.SemaphoreType.DMA,
            pltpu.SemaphoreType.DMA,
            pltpu.SemaphoreType.DMA,
            pltpu.SemaphoreType.DMA,
            pltpu.SemaphoreType.DMA,
            pltpu.SemaphoreType.DMA,
            pltpu.SemaphoreType.DMA,
        ],
    )


EDGE_G = 4


def _make_edge_agg(Epad, Npad, D):
    NW = NC * NS
    G = EDGE_G
    KPT = Epad // (NW * CH)
    NG = KPT // G
    NB = NG // 3
    rows_total = Epad // CH
    NPT = Npad // NS
    ZR = CH
    nz_full, nz_tail = divmod(NPT, ZR)

    def body(h, srcm, dstm, out, agg, sidx, didx, gbuf, zbuf,
             si0, si1, si2, sg0, sg1, sg2, ss0, ss1, ss2):
        semi = (si0, si1, si2)
        semg = (sg0, sg1, sg2)
        sems = (ss0, ss1, ss2)
        c = lax.axis_index("c")
        s = lax.axis_index("s")
        wid = c * NS + s
        zero = jnp.zeros((LANE,), jnp.float32)
        for i in range(ZR):
            zbuf[i, :] = zero
        row0 = s * NPT
        for k in range(nz_full):
            pltpu.async_copy(zbuf, agg.at[pl.ds(row0 + k * ZR, ZR)], si0)
        if nz_tail:
            pltpu.async_copy(zbuf.at[pl.ds(0, nz_tail)],
                            agg.at[pl.ds(row0 + nz_full * ZR, nz_tail)], si0)
        for k in range(nz_full):
            pltpu.make_async_copy(zbuf, agg.at[pl.ds(row0, ZR)], si0).wait()
        if nz_tail:
            pltpu.make_async_copy(zbuf.at[pl.ds(0, nz_tail)],
                                  agg.at[pl.ds(row0, nz_tail)], si0).wait()
        plsc.subcore_barrier()

        t0 = wid * KPT
        last = rows_total - G

        def gather(q, j2):
            return pltpu.async_copy(h.at[sidx.at[q].at[j2]],
                                    gbuf.at[q].at[j2], semg[q])

        def gather_wait(q, j2):
            pltpu.make_async_copy(h.at[sidx.at[q].at[j2]],
                                  gbuf.at[q].at[j2], semg[q]).wait()

        def scatter(q, j2):
            return pltpu.async_copy(gbuf.at[q].at[j2],
                                    agg.at[didx.at[q].at[j2]], sems[q],
                                    add=True)

        def scatter_wait(q, j2):
            pltpu.make_async_copy(gbuf.at[q].at[j2],
                                  agg.at[didx.at[q].at[j2]], sems[q]).wait()

        def idx_load(q, off):
            pltpu.async_copy(srcm.at[pl.ds(off, G)], sidx.at[q], semi[q])
            pltpu.async_copy(dstm.at[pl.ds(off, G)], didx.at[q], semi[q])

        def idx_wait(q):
            pltpu.make_async_copy(srcm.at[pl.ds(t0, G)], sidx.at[q],
                                  semi[q]).wait()
            pltpu.make_async_copy(dstm.at[pl.ds(t0, G)], didx.at[q],
                                  semi[q]).wait()

        pltpu.sync_copy(srcm.at[pl.ds(t0, G)], sidx.at[0])
        pltpu.sync_copy(dstm.at[pl.ds(t0, G)], didx.at[0])
        for j2 in range(G):
            gather(0, j2)
        idx_load(1, t0 + G)
        for j2 in range(G):
            pltpu.async_copy(zbuf, agg.at[didx.at[0].at[j2]], sems[2],
                             add=True)

        def block(ib, carry):
            jbase = ib * 3
            for k in range(3):
                jj = jbase + k
                q, q1, q2 = k, (k + 1) % 3, (k + 2) % 3
                for j2 in range(G):
                    gather_wait(q, j2)
                for j2 in range(G):
                    scatter(q, j2)
                idx_wait(q1)
                for j2 in range(G):
                    scatter_wait(q2, j2)
                for j2 in range(G):
                    gather(q1, j2)
                idx_load(q2, jnp.minimum(t0 + (jj + 2) * G, last))
            return carry

        lax.fori_loop(0, NB, block, 0)

        qlast = (NG - 1) % 3
        for j2 in range(G):
            scatter_wait(qlast, j2)
        for j2 in range(G):
            gather_wait(NG % 3, j2)
        idx_wait((NG + 1) % 3)

        plsc.subcore_barrier()
        pltpu.sync_copy(agg.at[pl.ds(row0, NPT)], out.at[c, pl.ds(row0, NPT)])

    return pl.kernel(
        body,
        out_type=jax.ShapeDtypeStruct((NC, Npad, D), jnp.float32),
        mesh=_sc_mesh(),
        compiler_params=pltpu.CompilerParams(use_tc_tiling_on_sc=False),
        scratch_types=[
            pltpu.VMEM_SHARED((Npad, D), jnp.float32),
            pltpu.VMEM((3, G, CH), jnp.int32),
            pltpu.VMEM((3, G, CH), jnp.int32),
            pltpu.VMEM((3, G, CH, D), jnp.float32),
            pltpu.VMEM((ZR, D), jnp.float32),
            pltpu.SemaphoreType.DMA,
            pltpu.SemaphoreType.DMA,
            pltpu.SemaphoreType.DMA,
            pltpu.SemaphoreType.DMA,
            pltpu.SemaphoreType.DMA,
            pltpu.SemaphoreType.DMA,
            pltpu.SemaphoreType.DMA,
            pltpu.SemaphoreType.DMA,
            pltpu.SemaphoreType.DMA,
        ],
    )


def _embed_tc(xg_p, degm, a_avg, degW_t, degb_t, lng_t, lnb_t, NP, BP):
    DB = BP * 8 // 128

    def body(xg_ref, d_ref, av_ref, w_ref, b_ref, g_ref, bb_ref, o_ref):
        dl = jnp.log1p(jnp.clip(d_ref[...], 0.0, 1e6))
        c_iota = lax.broadcasted_iota(jnp.int32, (128, 128), 0)
        l_iota = lax.broadcasted_iota(jnp.int32, (128, 128), 1)
        parts = []
        for k in range(16):
            wk = (c_iota == 8 * k + l_iota // 16).astype(jnp.float32)
            parts.append(jnp.dot(dl, wk, preferred_element_type=jnp.float32))
        dexp = jnp.stack(parts, axis=1).reshape(BP, 128)
        x = xg_ref[...] + dexp * w_ref[...] + b_ref[...]
        av = av_ref[...]
        m = jnp.dot(x, av, preferred_element_type=jnp.float32)
        xc = x - m
        v = jnp.dot(xc * xc, av, preferred_element_type=jnp.float32)
        y = xc / jnp.sqrt(v + 1e-5) * g_ref[...] + bb_ref[...]
        o_ref[...] = jax.nn.gelu(y)

    row = pl.BlockSpec((BP, 128), lambda i: (i, 0))
    vec = pl.BlockSpec((1, 128), lambda i: (0, 0))
    return pl.pallas_call(
        body,
        grid=(NP // BP,),
        in_specs=[row, pl.BlockSpec((DB, 128), lambda i: (i, 0)),
                  pl.BlockSpec((128, 128), lambda i: (0, 0)), vec, vec, vec,
                  vec],
        out_specs=row,
        out_shape=jax.ShapeDtypeStruct((NP, 128), jnp.float32),
    )(xg_p, degm, a_avg, degW_t, degb_t, lng_t, lnb_t)


def _gin_tc(h_p, aggpair_p, W1b, b1t, W2b, b2t, eps, NP, BP, N, final=None):
    def mlp(h_ref, agg_ref, w1, b1r, w2, b2r, eps_ref):
        hh = h_ref[...]
        agg = agg_ref[0] + agg_ref[1]
        z = (1.0 + eps_ref[0, 0]) * hh + agg
        z = jax.nn.gelu(jnp.dot(z, w1[...], preferred_element_type=jnp.float32)
                        + b1r[...])
        z = jnp.dot(z, w2[...], preferred_element_type=jnp.float32) + b2r[...]
        return z + hh

    row = pl.BlockSpec((BP, 128), lambda i: (i, 0))
    vec = pl.BlockSpec((1, 128), lambda i: (0, 0))
    mat = pl.BlockSpec((128, 128), lambda i: (0, 0))
    agg_spec = pl.BlockSpec((2, BP, 128), lambda i: (0, i, 0))
    scal = pl.BlockSpec((1, 1), lambda i: (0, 0))

    if final is None:
        def body(h_ref, agg_ref, w1, b1r, w2, b2r, eps_ref, o_ref):
            o_ref[...] = mlp(h_ref, agg_ref, w1, b1r, w2, b2r, eps_ref)

        return pl.pallas_call(
            body,
            grid=(NP // BP,),
            in_specs=[row, agg_spec, mat, vec, mat, vec, scal],
            out_specs=row,
            out_shape=jax.ShapeDtypeStruct((NP, 128), jnp.float32),
        )(h_p, aggpair_p, W1b, b1t, W2b, b2t, eps.reshape(1, 1))

    embed_p, alpha, pool_scale = final

    def body(h_ref, agg_ref, w1, b1r, w2, b2r, eps_ref, ex_ref, al_ref, ps_ref,
             o_ref):
        h2 = mlp(h_ref, agg_ref, w1, b1r, w2, b2r, eps_ref)
        jk = h_ref[...] + h2
        gate = jax.nn.sigmoid(al_ref[0, 0])
        out = gate * jk + (1.0 - gate) * ex_ref[...]
        o_ref[...] = out * jax.nn.softplus(ps_ref[0, 0])

    return pl.pallas_call(
        body,
        grid=(NP // BP,),
        in_specs=[row, agg_spec, mat, vec, mat, vec, scal, row, scal, scal],
        out_specs=row,
        out_shape=jax.ShapeDtypeStruct((NP, 128), jnp.float32),
    )(h_p, aggpair_p, W1b, b1t, W2b, b2t, eps.reshape(1, 1), embed_p,
      alpha.reshape(1, 1), pool_scale.reshape(1, 1))


def kernel(vertex_ids, labels, degree, edge_index, id_emb, label_emb, deg_W,
           deg_b, ln_g, ln_b, W1_0, b1_0, W2_0, b2_0, eps_0, W1_1, b1_1, W2_1,
           b2_1, eps_1, alpha, pool_scale):
    N, D = id_emb.shape
    L = label_emb.shape[0]
    E = edge_index.shape[1]
    NW = NC * NS
    unit_v = NW * CH * 3
    Nvpad = ((N + unit_v - 1) // unit_v) * unit_v
    NP = Nvpad // 8
    BP = NP // 6

    i8 = jnp.eye(8, dtype=jnp.float32)
    W1b_0 = jnp.kron(i8, W1_0)
    W2b_0 = jnp.kron(i8, W2_0)
    W1b_1 = jnp.kron(i8, W1_1)
    W2b_1 = jnp.kron(i8, W2_1)
    b1t_0 = jnp.tile(b1_0, 8).reshape(1, 128)
    b2t_0 = jnp.tile(b2_0, 8).reshape(1, 128)
    b1t_1 = jnp.tile(b1_1, 8).reshape(1, 128)
    b2t_1 = jnp.tile(b2_1, 8).reshape(1, 128)
    lng_t = jnp.tile(ln_g, 8).reshape(1, 128)
    lnb_t = jnp.tile(ln_b, 8).reshape(1, 128)
    degW_t = jnp.tile(deg_W, 8).reshape(1, 128)
    degb_t = jnp.tile(deg_b, 8).reshape(1, 128)
    a_avg = jnp.kron(i8, jnp.full((D, D), 1.0 / D, jnp.float32))
    degm = jnp.concatenate(
        [degree, jnp.zeros((Nvpad - N,), jnp.float32)]).reshape(-1, 128)

    padv = Nvpad - N
    fill = jnp.arange(padv, dtype=jnp.int32)
    vidm = jnp.concatenate([vertex_ids.astype(jnp.int32), fill % N]).reshape(-1, CH)
    labm = jnp.concatenate([labels.astype(jnp.int32), fill % L]).reshape(-1, CH)
    xg_p = _make_embed_gather(Nvpad, D)(id_emb, label_emb, vidm, labm)

    embed_p = _embed_tc(xg_p, degm, a_avg, degW_t, degb_t, lng_t, lnb_t,
                        NP, BP)

    unit_e = NW * CH * (3 * EDGE_G)
    Epad = ((E + unit_e - 1) // unit_e) * unit_e
    pade = Epad - E
    trash = 16
    Npad = N + trash
    fe = jnp.arange(pade, dtype=jnp.int32)
    srcm = jnp.concatenate([edge_index[0].astype(jnp.int32), fe % N]).reshape(-1, CH)
    dstm = jnp.concatenate([edge_index[1].astype(jnp.int32), N + fe % trash]).reshape(-1, CH)

    edge_agg = _make_edge_agg(Epad, Npad, D)

    agg0_p = edge_agg(embed_p.reshape(-1, D), srcm, dstm).reshape(NC, -1, 128)
    h1_p = _gin_tc(embed_p, agg0_p, W1b_0, b1t_0, W2b_0, b2t_0, eps_0, NP, BP, N)

    agg1_p = edge_agg(h1_p.reshape(-1, D), srcm, dstm).reshape(NC, -1, 128)
    out_p = _gin_tc(h1_p, agg1_p, W1b_1, b1t_1, W2b_1, b2t_1, eps_1, NP, BP, N,
                    final=(embed_p, alpha, pool_scale))
    return out_p.reshape(-1, D)[:N]


# speedup vs baseline: 1.1864x; 1.0555x over previous
import jax
import jax.numpy as jnp
from jax import lax
from jax.experimental import pallas as pl
from jax.experimental.pallas import tpu as pltpu
from jax.experimental.pallas import tpu_sc as plsc

NC = 2
NS = 16
LANE = 16
CH = 128


def _sc_mesh():
    return plsc.VectorSubcoreMesh(
        core_axis_name="c", subcore_axis_name="s", num_cores=NC, num_subcores=NS)


def _make_embed_gather(Nvpad, D, L):
    NW = NC * NS
    KV = Nvpad // (NW * CH)
    rows_max = Nvpad // CH - 1

    PR = CH // 8

    def body(id_emb, label_emb, vidm, labm, out, vidx, lidx, gbuf, ltab, obuf,
             si0, si1, si2, sg0, sg1, sg2, sw0, sw1, sw2):
        semi = (si0, si1, si2)
        semg = (sg0, sg1, sg2)
        semw = (sw0, sw1, sw2)
        c = lax.axis_index("c")
        s = lax.axis_index("s")
        wid = c * NS + s
        t0 = wid * KV
        iota16 = lax.iota(jnp.int32, LANE)
        pltpu.sync_copy(label_emb, ltab)

        def idx_load(q, r):
            pltpu.async_copy(vidm.at[r], vidx.at[q], semi[q])
            pltpu.async_copy(labm.at[r], lidx.at[q], semi[q])

        def idx_wait(q):
            pltpu.make_async_copy(vidm.at[t0], vidx.at[q], semi[q]).wait()
            pltpu.make_async_copy(labm.at[t0], lidx.at[q], semi[q]).wait()

        def gathers(q):
            pltpu.async_copy(id_emb.at[vidx.at[q]], gbuf.at[q], semg[q])

        def gathers_wait(q):
            pltpu.make_async_copy(id_emb.at[vidx.at[q]], gbuf.at[q],
                                  semg[q]).wait()

        trash0 = Nvpad // 8

        def wout_wait(q):
            pltpu.make_async_copy(obuf.at[q], out.at[pl.ds(trash0, PR)],
                                  semw[q]).wait()

        pltpu.sync_copy(vidm.at[t0], vidx.at[0])
        pltpu.sync_copy(labm.at[t0], lidx.at[0])
        gathers(0)
        idx_load(1, t0 + 1)
        for q in range(3):
            pltpu.async_copy(obuf.at[q], out.at[pl.ds(trash0, PR)], semw[q])

        def block(ib, carry):
            ibase = ib * 3
            for k in range(3):
                i = ibase + k
                q, q1, q2 = k, (k + 1) % 3, (k + 2) % 3
                gathers_wait(q)
                idx_wait(q1)
                gathers(q1)
                wout_wait(q)
                for jj in range(CH):
                    lab = plsc.load_gather(
                        lidx, [jnp.full((LANE,), q, jnp.int32),
                               jnp.full((LANE,), jj, jnp.int32)])
                    lrow = plsc.load_gather(ltab, [lab, iota16])
                    obuf[q, jj // 8, pl.ds((jj % 8) * LANE, LANE)] = (
                        gbuf[q, jj, :] + lrow)
                pltpu.async_copy(obuf.at[q],
                                 out.at[pl.ds((t0 + i) * PR, PR)], semw[q])
                idx_load(q2, jnp.minimum(t0 + i + 2, rows_max))
            return carry

        lax.fori_loop(0, KV // 3, block, 0)

        gathers_wait(KV % 3)
        idx_wait((KV + 1) % 3)
        for q in range(3):
            wout_wait(q)

    return pl.kernel(
        body,
        out_type=jax.ShapeDtypeStruct((Nvpad // 8 + CH // 8, 128), jnp.float32),
        mesh=_sc_mesh(),
        compiler_params=pltpu.CompilerParams(use_tc_tiling_on_sc=False,
                                             needs_layout_passes=False),
        scratch_types=[
            pltpu.VMEM((3, CH), jnp.int32),
            pltpu.VMEM((3, CH), jnp.int32),
            pltpu.VMEM((3, CH, D), jnp.float32),
            pltpu.VMEM((L, D), jnp.float32),
            pltpu.VMEM((3, CH // 8, 128), jnp.float32),
            pltpu.SemaphoreType.DMA,
            pltpu.SemaphoreType.DMA,
            pltpu.SemaphoreType.DMA,
            pltpu.SemaphoreType.DMA,
            pltpu.SemaphoreType.DMA,
            pltpu.SemaphoreType.DMA,
            pltpu.SemaphoreType.DMA,
            pltpu.SemaphoreType.DMA,
            pltpu.SemaphoreType.DMA,
        ],
    )


EDGE_G = 4


def _make_edge_agg(Epad, Npad, D):
    NW = NC * NS
    G = EDGE_G
    KPT = Epad // (NW * CH)
    NG = KPT // G
    NB = NG // 3
    rows_total = Epad // CH
    NPT = Npad // NS
    ZR = CH
    nz_full, nz_tail = divmod(NPT, ZR)

    def body(h, srcm, dstm, out, agg, sidx, didx, gbuf, zbuf,
             si0, si1, si2, sg0, sg1, sg2, ss0, ss1, ss2):
        semi = (si0, si1, si2)
        semg = (sg0, sg1, sg2)
        sems = (ss0, ss1, ss2)
        c = lax.axis_index("c")
        s = lax.axis_index("s")
        wid = c * NS + s
        zero = jnp.zeros((LANE,), jnp.float32)
        for i in range(ZR):
            zbuf[i, :] = zero
        row0 = s * NPT
        for k in range(nz_full):
            pltpu.async_copy(zbuf, agg.at[pl.ds(row0 + k * ZR, ZR)], si0)
        if nz_tail:
            pltpu.async_copy(zbuf.at[pl.ds(0, nz_tail)],
                            agg.at[pl.ds(row0 + nz_full * ZR, nz_tail)], si0)
        for k in range(nz_full):
            pltpu.make_async_copy(zbuf, agg.at[pl.ds(row0, ZR)], si0).wait()
        if nz_tail:
            pltpu.make_async_copy(zbuf.at[pl.ds(0, nz_tail)],
                                  agg.at[pl.ds(row0, nz_tail)], si0).wait()
        plsc.subcore_barrier()

        t0 = wid * KPT
        last = rows_total - G

        def gather(q, j2):
            return pltpu.async_copy(h.at[sidx.at[q].at[j2]],
                                    gbuf.at[q].at[j2], semg[q])

        def gather_wait(q, j2):
            pltpu.make_async_copy(h.at[sidx.at[q].at[j2]],
                                  gbuf.at[q].at[j2], semg[q]).wait()

        def scatter(q, j2):
            return pltpu.async_copy(gbuf.at[q].at[j2],
                                    agg.at[didx.at[q].at[j2]], sems[q],
                                    add=True)

        def scatter_wait(q, j2):
            pltpu.make_async_copy(gbuf.at[q].at[j2],
                                  agg.at[didx.at[q].at[j2]], sems[q]).wait()

        def idx_load(q, off):
            pltpu.async_copy(srcm.at[pl.ds(off, G)], sidx.at[q], semi[q])
            pltpu.async_copy(dstm.at[pl.ds(off, G)], didx.at[q], semi[q])

        def idx_wait(q):
            pltpu.make_async_copy(srcm.at[pl.ds(t0, G)], sidx.at[q],
                                  semi[q]).wait()
            pltpu.make_async_copy(dstm.at[pl.ds(t0, G)], didx.at[q],
                                  semi[q]).wait()

        pltpu.sync_copy(srcm.at[pl.ds(t0, G)], sidx.at[0])
        pltpu.sync_copy(dstm.at[pl.ds(t0, G)], didx.at[0])
        for j2 in range(G):
            gather(0, j2)
        idx_load(1, t0 + G)
        for j2 in range(G):
            pltpu.async_copy(zbuf, agg.at[didx.at[0].at[j2]], sems[2],
                             add=True)

        def block(ib, carry):
            jbase = ib * 3
            for k in range(3):
                jj = jbase + k
                q, q1, q2 = k, (k + 1) % 3, (k + 2) % 3
                for j2 in range(G):
                    gather_wait(q, j2)
                for j2 in range(G):
                    scatter(q, j2)
                idx_wait(q1)
                for j2 in range(G):
                    scatter_wait(q2, j2)
                for j2 in range(G):
                    gather(q1, j2)
                idx_load(q2, jnp.minimum(t0 + (jj + 2) * G, last))
            return carry

        lax.fori_loop(0, NB, block, 0)

        qlast = (NG - 1) % 3
        for j2 in range(G):
            scatter_wait(qlast, j2)
        for j2 in range(G):
            gather_wait(NG % 3, j2)
        idx_wait((NG + 1) % 3)

        plsc.subcore_barrier()
        pltpu.sync_copy(agg.at[pl.ds(row0, NPT)], out.at[c, pl.ds(row0, NPT)])

    return pl.kernel(
        body,
        out_type=jax.ShapeDtypeStruct((NC, Npad, D), jnp.float32),
        mesh=_sc_mesh(),
        compiler_params=pltpu.CompilerParams(use_tc_tiling_on_sc=False),
        scratch_types=[
            pltpu.VMEM_SHARED((Npad, D), jnp.float32),
            pltpu.VMEM((3, G, CH), jnp.int32),
            pltpu.VMEM((3, G, CH), jnp.int32),
            pltpu.VMEM((3, G, CH, D), jnp.float32),
            pltpu.VMEM((ZR, D), jnp.float32),
            pltpu.SemaphoreType.DMA,
            pltpu.SemaphoreType.DMA,
            pltpu.SemaphoreType.DMA,
            pltpu.SemaphoreType.DMA,
            pltpu.SemaphoreType.DMA,
            pltpu.SemaphoreType.DMA,
            pltpu.SemaphoreType.DMA,
            pltpu.SemaphoreType.DMA,
            pltpu.SemaphoreType.DMA,
        ],
    )


def _embed_tc(xg_p, degm, a_avg, degW_t, degb_t, lng_t, lnb_t, NP, BP):
    DB = BP * 8 // 128

    def body(xg_ref, d_ref, av_ref, w_ref, b_ref, g_ref, bb_ref, o_ref):
        dl = jnp.log1p(jnp.clip(d_ref[...], 0.0, 1e6))
        c_iota = lax.broadcasted_iota(jnp.int32, (128, 128), 0)
        l_iota = lax.broadcasted_iota(jnp.int32, (128, 128), 1)
        parts = []
        for k in range(16):
            wk = (c_iota == 8 * k + l_iota // 16).astype(jnp.float32)
            parts.append(jnp.dot(dl, wk, preferred_element_type=jnp.float32))
        dexp = jnp.stack(parts, axis=1).reshape(BP, 128)
        x = xg_ref[...] + dexp * w_ref[...] + b_ref[...]
        av = av_ref[...]
        m = jnp.dot(x, av, preferred_element_type=jnp.float32)
        xc = x - m
        v = jnp.dot(xc * xc, av, preferred_element_type=jnp.float32)
        y = xc / jnp.sqrt(v + 1e-5) * g_ref[...] + bb_ref[...]
        o_ref[...] = jax.nn.gelu(y)

    row = pl.BlockSpec((BP, 128), lambda i: (i, 0))
    vec = pl.BlockSpec((1, 128), lambda i: (0, 0))
    return pl.pallas_call(
        body,
        grid=(NP // BP,),
        in_specs=[row, pl.BlockSpec((DB, 128), lambda i: (i, 0)),
                  pl.BlockSpec((128, 128), lambda i: (0, 0)), vec, vec, vec,
                  vec],
        out_specs=row,
        out_shape=jax.ShapeDtypeStruct((NP, 128), jnp.float32),
    )(xg_p, degm, a_avg, degW_t, degb_t, lng_t, lnb_t)


def _gin_tc(h_p, aggpair_p, W1b, b1t, W2b, b2t, eps, NP, BP, N, final=None):
    def mlp(h_ref, agg_ref, w1, b1r, w2, b2r, eps_ref):
        hh = h_ref[...]
        agg = agg_ref[0] + agg_ref[1]
        z = (1.0 + eps_ref[0, 0]) * hh + agg
        z = jax.nn.gelu(jnp.dot(z, w1[...], preferred_element_type=jnp.float32)
                        + b1r[...])
        z = jnp.dot(z, w2[...], preferred_element_type=jnp.float32) + b2r[...]
        return z + hh

    row = pl.BlockSpec((BP, 128), lambda i: (i, 0))
    vec = pl.BlockSpec((1, 128), lambda i: (0, 0))
    mat = pl.BlockSpec((128, 128), lambda i: (0, 0))
    agg_spec = pl.BlockSpec((2, BP, 128), lambda i: (0, i, 0))
    scal = pl.BlockSpec((1, 1), lambda i: (0, 0))

    if final is None:
        def body(h_ref, agg_ref, w1, b1r, w2, b2r, eps_ref, o_ref):
            o_ref[...] = mlp(h_ref, agg_ref, w1, b1r, w2, b2r, eps_ref)

        return pl.pallas_call(
            body,
            grid=(NP // BP,),
            in_specs=[row, agg_spec, mat, vec, mat, vec, scal],
            out_specs=row,
            out_shape=jax.ShapeDtypeStruct((NP, 128), jnp.float32),
        )(h_p, aggpair_p, W1b, b1t, W2b, b2t, eps.reshape(1, 1))

    embed_p, alpha, pool_scale = final

    def body(h_ref, agg_ref, w1, b1r, w2, b2r, eps_ref, ex_ref, al_ref, ps_ref,
             o_ref):
        h2 = mlp(h_ref, agg_ref, w1, b1r, w2, b2r, eps_ref)
        jk = h_ref[...] + h2
        gate = jax.nn.sigmoid(al_ref[0, 0])
        out = gate * jk + (1.0 - gate) * ex_ref[...]
        o_ref[...] = out * jax.nn.softplus(ps_ref[0, 0])

    return pl.pallas_call(
        body,
        grid=(NP // BP,),
        in_specs=[row, agg_spec, mat, vec, mat, vec, scal, row, scal, scal],
        out_specs=row,
        out_shape=jax.ShapeDtypeStruct((NP, 128), jnp.float32),
    )(h_p, aggpair_p, W1b, b1t, W2b, b2t, eps.reshape(1, 1), embed_p,
      alpha.reshape(1, 1), pool_scale.reshape(1, 1))


def kernel(vertex_ids, labels, degree, edge_index, id_emb, label_emb, deg_W,
           deg_b, ln_g, ln_b, W1_0, b1_0, W2_0, b2_0, eps_0, W1_1, b1_1, W2_1,
           b2_1, eps_1, alpha, pool_scale):
    N, D = id_emb.shape
    L = label_emb.shape[0]
    E = edge_index.shape[1]
    NW = NC * NS
    unit_v = NW * CH * 3
    Nvpad = ((N + unit_v - 1) // unit_v) * unit_v
    NP = Nvpad // 8
    BP = NP // 6

    i8 = jnp.eye(8, dtype=jnp.float32)
    W1b_0 = jnp.kron(i8, W1_0)
    W2b_0 = jnp.kron(i8, W2_0)
    W1b_1 = jnp.kron(i8, W1_1)
    W2b_1 = jnp.kron(i8, W2_1)
    b1t_0 = jnp.tile(b1_0, 8).reshape(1, 128)
    b2t_0 = jnp.tile(b2_0, 8).reshape(1, 128)
    b1t_1 = jnp.tile(b1_1, 8).reshape(1, 128)
    b2t_1 = jnp.tile(b2_1, 8).reshape(1, 128)
    lng_t = jnp.tile(ln_g, 8).reshape(1, 128)
    lnb_t = jnp.tile(ln_b, 8).reshape(1, 128)
    degW_t = jnp.tile(deg_W, 8).reshape(1, 128)
    degb_t = jnp.tile(deg_b, 8).reshape(1, 128)
    a_avg = jnp.kron(i8, jnp.full((D, D), 1.0 / D, jnp.float32))
    degm = jnp.concatenate(
        [degree, jnp.zeros((Nvpad - N,), jnp.float32)]).reshape(-1, 128)

    padv = Nvpad - N
    fill = jnp.arange(padv, dtype=jnp.int32)
    vidm = jnp.concatenate([vertex_ids.astype(jnp.int32), fill % N]).reshape(-1, CH)
    labm = jnp.concatenate([labels.astype(jnp.int32), fill % L]).reshape(-1, CH)
    xg_p = _make_embed_gather(Nvpad, D, L)(id_emb, label_emb, vidm, labm)

    embed_p = _embed_tc(xg_p, degm, a_avg, degW_t, degb_t, lng_t, lnb_t,
                        NP, BP)

    unit_e = NW * CH * (3 * EDGE_G)
    Epad = ((E + unit_e - 1) // unit_e) * unit_e
    pade = Epad - E
    trash = 16
    Npad = N + trash
    fe = jnp.arange(pade, dtype=jnp.int32)
    srcm = jnp.concatenate([edge_index[0].astype(jnp.int32), fe % N]).reshape(-1, CH)
    dstm = jnp.concatenate([edge_index[1].astype(jnp.int32), N + fe % trash]).reshape(-1, CH)

    edge_agg = _make_edge_agg(Epad, Npad, D)

    agg0_p = edge_agg(embed_p.reshape(-1, D), srcm, dstm).reshape(NC, -1, 128)
    h1_p = _gin_tc(embed_p, agg0_p, W1b_0, b1t_0, W2b_0, b2t_0, eps_0, NP, BP, N)

    agg1_p = edge_agg(h1_p.reshape(-1, D), srcm, dstm).reshape(NC, -1, 128)
    out_p = _gin_tc(h1_p, agg1_p, W1b_1, b1t_1, W2b_1, b2t_1, eps_1, NP, BP, N,
                    final=(embed_p, alpha, pool_scale))
    return out_p.reshape(-1, D)[:N]
